# baseline (speedup 1.0000x reference)
import functools

import jax
import jax.numpy as jnp
from jax import lax
from jax.experimental import pallas as pl
from jax.experimental.pallas import tpu as pltpu
from jax.experimental.pallas import tpu_sc as plsc

D = 32
NC, NS = 2, 16
NW = NC * NS
CB = 128
RB = D // 8
V = 1000000
W1 = 512
NBLK1 = V // W1
TROWS = W1 * D // CB
NT1 = 62


@jax.jit
def _detile_table(wt, wtail):
    mesh = plsc.VectorSubcoreMesh(core_axis_name="c", subcore_axis_name="s")

    @functools.partial(
        pl.kernel,
        out_type=jax.ShapeDtypeStruct((V * D // CB, CB), jnp.float32),
        mesh=mesh,
        scratch_types=[
            pltpu.VMEM((D, W1), jnp.float32),
            pltpu.VMEM((D, W1), jnp.float32),
            pltpu.VMEM((TROWS, CB), jnp.float32),
            pltpu.VMEM((TROWS, CB), jnp.float32),
            pltpu.SemaphoreType.DMA,
            pltpu.SemaphoreType.DMA,
            pltpu.SemaphoreType.DMA,
            pltpu.SemaphoreType.DMA,
        ],
        compiler_params=pltpu.CompilerParams(needs_layout_passes=False),
    )
    def k1(wt_hbm, wtail_hbm, out_hbm, blk0, blk1, tr0, tr1, rs0, rs1,
           ws0, ws1):
        blks, trs = [blk0, blk1], [tr0, tr1]
        rsems, wsems = [rs0, rs1], [ws0, ws1]
        wid = lax.axis_index("s") * NC + lax.axis_index("c")
        lane = lax.iota(jnp.int32, 16)
        ridx = [lane, lane + 16]

        @pl.when(wid == NW - 1)
        def _():
            pltpu.sync_copy(wtail_hbm, out_hbm.at[pl.ds(V * D // CB - 16, 16)])

        def issue_read(tl, s):
            t = tl * NW + wid

            @pl.when(t < NBLK1)
            def _():
                pltpu.async_copy(
                    wt_hbm.at[:, pl.ds(t * W1, W1)], blks[s], rsems[s]
                )

        issue_read(jnp.int32(0), 0)

        def transpose(src, dst):
            def tbody(it, c):
                for j in range(8):
                    i = it * 8 + j
                    iv = jnp.full((16,), 0, jnp.int32) + i
                    for h in range(2):
                        n0 = i * D + h * 16
                        vals = plsc.load_gather(src, [ridx[h], iv])
                        dst[n0 // CB, pl.ds(n0 % CB, 16)] = vals
                return c

            lax.fori_loop(0, W1 // 8, tbody, 0)

        def outer(o, carry):
            for s in range(2):
                tl = o * 2 + s
                issue_read(tl + 1, 1 - s)
                t = tl * NW + wid

                @pl.when(t < NBLK1)
                def _():
                    pltpu.make_async_copy(
                        wt_hbm.at[:, pl.ds(0, W1)], blks[s], rsems[s]
                    ).wait()

                    @pl.when(tl >= 2)
                    def __():
                        pltpu.make_async_copy(
                            trs[s], out_hbm.at[pl.ds(0, TROWS)], wsems[s]
                        ).wait()

                    transpose(blks[s], trs[s])
                    pltpu.async_copy(
                        trs[s], out_hbm.at[pl.ds(t * TROWS, TROWS)], wsems[s]
                    )
            return carry

        lax.fori_loop(0, NT1 // 2, outer, 0)
        for s in range(2):
            tl_last = NT1 - 2 + s
            t = tl_last * NW + wid

            @pl.when(t < NBLK1)
            def _():
                pltpu.make_async_copy(
                    trs[s], out_hbm.at[pl.ds(0, TROWS)], wsems[s]
                ).wait()

    return k1(wt, wtail)


@functools.partial(jax.jit, static_argnums=(1, 2))
def _gather_rows(args, b, f):
    xt2, wlin = args
    nchunk = f * (b // CB)
    per_w = nchunk // NW

    mesh = plsc.VectorSubcoreMesh(core_axis_name="c", subcore_axis_name="s")

    @functools.partial(
        pl.kernel,
        out_type=jax.ShapeDtypeStruct((f, RB, b // CB, 8, CB), jnp.float32),
        mesh=mesh,
        scratch_types=[
            pltpu.VMEM((per_w, CB), jnp.int32),
            pltpu.VMEM((CB, D), jnp.float32),
            pltpu.VMEM((D, CB), jnp.float32),
            pltpu.SemaphoreType.DMA,
        ],
        compiler_params=pltpu.CompilerParams(
            use_tc_tiling_on_sc=False, needs_layout_passes=False
        ),
    )
    def k2(xt_hbm, w_hbm, out_hbm, idx_v, rows_v, tr_v, sem):
        wid = lax.axis_index("s") * NC + lax.axis_index("c")
        pltpu.sync_copy(xt_hbm.at[pl.ds(wid * per_w, per_w)], idx_v)
        lane = lax.iota(jnp.int32, 16)
        cidx = [lane + (c0 * 16) for c0 in range(CB // 16)]

        def chunk(t_local, carry):
            t = wid * per_w + t_local
            fi = t // (b // CB)
            cb = t % (b // CB)
            pltpu.async_copy(w_hbm.at[idx_v.at[t_local]], rows_v, sem).wait()
            for d in range(D):
                didx = jnp.full((16,), d, jnp.int32)
                for c0 in range(CB // 16):
                    tr_v[d, pl.ds(c0 * 16, 16)] = plsc.load_gather(
                        rows_v, [cidx[c0], didx]
                    )
            for rb in range(RB):
                pltpu.sync_copy(
                    tr_v.at[pl.ds(rb * 8, 8)], out_hbm.at[fi, rb, cb]
                )
            return carry

        lax.fori_loop(0, per_w, chunk, 0)

    return k2(xt2, wlin)


def kernel(x, weight):
    b, f = x.shape
    xt2 = x.T.astype(jnp.int32).reshape(f * (b // CB), CB)
    wtail = weight[V - 64:].reshape(16, CB)
    wlin = _detile_table(weight.T, wtail)
    y = _gather_rows((xt2, wlin.reshape(V, D)), b, f)
    return y.transpose(2, 4, 0, 1, 3).reshape(b, f, D)


# speedup vs baseline: 1.6550x; 1.6550x over previous
import functools

import jax
import jax.numpy as jnp
from jax import lax
from jax.experimental import pallas as pl
from jax.experimental.pallas import tpu as pltpu
from jax.experimental.pallas import tpu_sc as plsc

D = 32
NC, NS = 2, 16
NW = NC * NS
CB = 128
RB = D // 8
V = 1000000
W1 = 512
NBLK1 = V // W1
TROWS = W1 * D // CB
NT1 = 62


@jax.jit
def _detile_table(wt, wtail):
    mesh = plsc.VectorSubcoreMesh(core_axis_name="c", subcore_axis_name="s")

    @functools.partial(
        pl.kernel,
        out_type=jax.ShapeDtypeStruct((V * D // CB, CB), jnp.float32),
        mesh=mesh,
        scratch_types=[
            pltpu.VMEM((D, W1), jnp.float32),
            pltpu.VMEM((D, W1), jnp.float32),
            pltpu.VMEM((TROWS, CB), jnp.float32),
            pltpu.VMEM((TROWS, CB), jnp.float32),
            pltpu.SemaphoreType.DMA,
            pltpu.SemaphoreType.DMA,
            pltpu.SemaphoreType.DMA,
            pltpu.SemaphoreType.DMA,
        ],
        compiler_params=pltpu.CompilerParams(needs_layout_passes=False),
    )
    def k1(wt_hbm, wtail_hbm, out_hbm, blk0, blk1, tr0, tr1, rs0, rs1,
           ws0, ws1):
        blks, trs = [blk0, blk1], [tr0, tr1]
        rsems, wsems = [rs0, rs1], [ws0, ws1]
        wid = lax.axis_index("s") * NC + lax.axis_index("c")
        lane = lax.iota(jnp.int32, 16)
        ridx = [lane, lane + 16]

        @pl.when(wid == NW - 1)
        def _():
            pltpu.sync_copy(wtail_hbm, out_hbm.at[pl.ds(V * D // CB - 16, 16)])

        def issue_read(tl, s):
            t = tl * NW + wid

            @pl.when(t < NBLK1)
            def _():
                pltpu.async_copy(
                    wt_hbm.at[:, pl.ds(t * W1, W1)], blks[s], rsems[s]
                )

        issue_read(jnp.int32(0), 0)

        def transpose(src, dst):
            @plsc.parallel_loop(0, W1, step=1, unroll=8)
            def tbody(i):
                iv = jnp.full((16,), 0, jnp.int32) + i
                for h in range(2):
                    n0 = i * D + h * 16
                    vals = plsc.load_gather(src, [ridx[h], iv])
                    dst[n0 // CB, pl.ds(n0 % CB, 16)] = vals

        def outer(o, carry):
            for s in range(2):
                tl = o * 2 + s
                issue_read(tl + 1, 1 - s)
                t = tl * NW + wid

                @pl.when(t < NBLK1)
                def _():
                    pltpu.make_async_copy(
                        wt_hbm.at[:, pl.ds(0, W1)], blks[s], rsems[s]
                    ).wait()

                    @pl.when(tl >= 2)
                    def __():
                        pltpu.make_async_copy(
                            trs[s], out_hbm.at[pl.ds(0, TROWS)], wsems[s]
                        ).wait()

                    transpose(blks[s], trs[s])
                    pltpu.async_copy(
                        trs[s], out_hbm.at[pl.ds(t * TROWS, TROWS)], wsems[s]
                    )
            return carry

        lax.fori_loop(0, NT1 // 2, outer, 0)
        for s in range(2):
            tl_last = NT1 - 2 + s
            t = tl_last * NW + wid

            @pl.when(t < NBLK1)
            def _():
                pltpu.make_async_copy(
                    trs[s], out_hbm.at[pl.ds(0, TROWS)], wsems[s]
                ).wait()

    return k1(wt, wtail)


@functools.partial(jax.jit, static_argnums=(1, 2))
def _gather_rows(args, b, f):
    xt2, wlin = args
    nchunk = f * (b // CB)
    per_w = nchunk // NW

    mesh = plsc.VectorSubcoreMesh(core_axis_name="c", subcore_axis_name="s")

    @functools.partial(
        pl.kernel,
        out_type=jax.ShapeDtypeStruct((f, RB, b // CB, 8, CB), jnp.float32),
        mesh=mesh,
        scratch_types=[
            pltpu.VMEM((per_w, CB), jnp.int32),
            pltpu.VMEM((CB, D), jnp.float32),
            pltpu.VMEM((D, CB), jnp.float32),
            pltpu.SemaphoreType.DMA,
        ],
        compiler_params=pltpu.CompilerParams(
            use_tc_tiling_on_sc=False, needs_layout_passes=False
        ),
    )
    def k2(xt_hbm, w_hbm, out_hbm, idx_v, rows_v, tr_v, sem):
        wid = lax.axis_index("s") * NC + lax.axis_index("c")
        pltpu.sync_copy(xt_hbm.at[pl.ds(wid * per_w, per_w)], idx_v)
        lane = lax.iota(jnp.int32, 16)
        cidx = [lane + (c0 * 16) for c0 in range(CB // 16)]

        def chunk(t_local, carry):
            t = wid * per_w + t_local
            fi = t // (b // CB)
            cb = t % (b // CB)
            pltpu.async_copy(w_hbm.at[idx_v.at[t_local]], rows_v, sem).wait()
            @plsc.parallel_loop(0, D, step=1, unroll=8)
            def tbody(d):
                didx = jnp.full((16,), 0, jnp.int32) + d
                for c0 in range(CB // 16):
                    tr_v[d, pl.ds(c0 * 16, 16)] = plsc.load_gather(
                        rows_v, [cidx[c0], didx]
                    )
            for rb in range(RB):
                pltpu.sync_copy(
                    tr_v.at[pl.ds(rb * 8, 8)], out_hbm.at[fi, rb, cb]
                )
            return carry

        lax.fori_loop(0, per_w, chunk, 0)

    return k2(xt2, wlin)


def kernel(x, weight):
    b, f = x.shape
    xt2 = x.T.astype(jnp.int32).reshape(f * (b // CB), CB)
    wtail = weight[V - 64:].reshape(16, CB)
    wlin = _detile_table(weight.T, wtail)
    y = _gather_rows((xt2, wlin.reshape(V, D)), b, f)
    return y.transpose(2, 4, 0, 1, 3).reshape(b, f, D)


# speedup vs baseline: 1.9995x; 1.2081x over previous
import functools

import jax
import jax.numpy as jnp
from jax import lax
from jax.experimental import pallas as pl
from jax.experimental.pallas import tpu as pltpu
from jax.experimental.pallas import tpu_sc as plsc

D = 32
NC, NS = 2, 16
NW = NC * NS
CB = 128
RB = D // 8
V = 1000000
W1 = 512
NBLK1 = V // W1
TROWS = W1 * D // CB
NT1 = 62


@jax.jit
def _detile_table(wt, wtail):
    mesh = plsc.VectorSubcoreMesh(core_axis_name="c", subcore_axis_name="s")

    @functools.partial(
        pl.kernel,
        out_type=jax.ShapeDtypeStruct((V * D // CB, CB), jnp.float32),
        mesh=mesh,
        scratch_types=[
            pltpu.VMEM((D, W1), jnp.float32),
            pltpu.VMEM((D, W1), jnp.float32),
            pltpu.VMEM((TROWS, CB), jnp.float32),
            pltpu.VMEM((TROWS, CB), jnp.float32),
            pltpu.SemaphoreType.DMA,
            pltpu.SemaphoreType.DMA,
            pltpu.SemaphoreType.DMA,
            pltpu.SemaphoreType.DMA,
        ],
        compiler_params=pltpu.CompilerParams(needs_layout_passes=False),
    )
    def k1(wt_hbm, wtail_hbm, out_hbm, blk0, blk1, tr0, tr1, rs0, rs1,
           ws0, ws1):
        blks, trs = [blk0, blk1], [tr0, tr1]
        rsems, wsems = [rs0, rs1], [ws0, ws1]
        wid = lax.axis_index("s") * NC + lax.axis_index("c")
        lane = lax.iota(jnp.int32, 16)
        ridx = [lane, lane + 16]

        @pl.when(wid == NW - 1)
        def _():
            pltpu.sync_copy(wtail_hbm, out_hbm.at[pl.ds(V * D // CB - 16, 16)])

        def issue_read(tl, s):
            t = tl * NW + wid

            @pl.when(t < NBLK1)
            def _():
                pltpu.async_copy(
                    wt_hbm.at[:, pl.ds(t * W1, W1)], blks[s], rsems[s]
                )

        issue_read(jnp.int32(0), 0)

        def transpose(src, dst):
            @plsc.parallel_loop(0, W1, step=1, unroll=16)
            def tbody(i):
                iv = jnp.full((16,), 0, jnp.int32) + i
                for h in range(2):
                    n0 = i * D + h * 16
                    vals = plsc.load_gather(src, [ridx[h], iv])
                    dst[n0 // CB, pl.ds(n0 % CB, 16)] = vals

        def outer(o, carry):
            for s in range(2):
                tl = o * 2 + s
                issue_read(tl + 1, 1 - s)
                t = tl * NW + wid

                @pl.when(t < NBLK1)
                def _():
                    pltpu.make_async_copy(
                        wt_hbm.at[:, pl.ds(0, W1)], blks[s], rsems[s]
                    ).wait()

                    @pl.when(tl >= 2)
                    def __():
                        pltpu.make_async_copy(
                            trs[s], out_hbm.at[pl.ds(0, TROWS)], wsems[s]
                        ).wait()

                    transpose(blks[s], trs[s])
                    pltpu.async_copy(
                        trs[s], out_hbm.at[pl.ds(t * TROWS, TROWS)], wsems[s]
                    )
            return carry

        lax.fori_loop(0, NT1 // 2, outer, 0)
        for s in range(2):
            tl_last = NT1 - 2 + s
            t = tl_last * NW + wid

            @pl.when(t < NBLK1)
            def _():
                pltpu.make_async_copy(
                    trs[s], out_hbm.at[pl.ds(0, TROWS)], wsems[s]
                ).wait()

    return k1(wt, wtail)


@functools.partial(jax.jit, static_argnums=(1, 2))
def _gather_rows(args, b, f):
    xt2, wlin = args
    nchunk = f * (b // CB)
    per_w = nchunk // NW

    mesh = plsc.VectorSubcoreMesh(core_axis_name="c", subcore_axis_name="s")

    @functools.partial(
        pl.kernel,
        out_type=jax.ShapeDtypeStruct((f, RB, b // CB, 8, CB), jnp.float32),
        mesh=mesh,
        scratch_types=[
            pltpu.VMEM((per_w, CB), jnp.int32),
            pltpu.VMEM((CB, D), jnp.float32),
            pltpu.VMEM((CB, D), jnp.float32),
            pltpu.VMEM((D, CB), jnp.float32),
            pltpu.VMEM((D, CB), jnp.float32),
            pltpu.SemaphoreType.DMA,
            pltpu.SemaphoreType.DMA,
            pltpu.SemaphoreType.DMA,
            pltpu.SemaphoreType.DMA,
        ],
        compiler_params=pltpu.CompilerParams(
            use_tc_tiling_on_sc=False, needs_layout_passes=False
        ),
    )
    def k2(xt_hbm, w_hbm, out_hbm, idx_v, rows0, rows1, tr0, tr1,
           gs0, gs1, ws0, ws1):
        rows, trs = [rows0, rows1], [tr0, tr1]
        gsems, wsems = [gs0, gs1], [ws0, ws1]
        wid = lax.axis_index("s") * NC + lax.axis_index("c")
        pltpu.sync_copy(xt_hbm.at[pl.ds(wid * per_w, per_w)], idx_v)
        lane = lax.iota(jnp.int32, 16)
        cidx = [lane + (c0 * 16) for c0 in range(CB // 16)]

        def issue_gather(tl, s):
            @pl.when(tl < per_w)
            def _():
                pltpu.async_copy(
                    w_hbm.at[idx_v.at[tl]], rows[s], gsems[s]
                )

        issue_gather(jnp.int32(0), 0)

        def outer(o, carry):
            for s in range(2):
                tl = o * 2 + s
                issue_gather(tl + 1, 1 - s)
                t = wid * per_w + tl
                fi = t // (b // CB)
                cb = t % (b // CB)
                pltpu.make_async_copy(
                    w_hbm.at[idx_v.at[0]], rows[s], gsems[s]
                ).wait()

                @pl.when(tl >= 2)
                def _():
                    for rb in range(RB):
                        pltpu.make_async_copy(
                            trs[s].at[pl.ds(rb * 8, 8)],
                            out_hbm.at[0, rb, 0],
                            wsems[s],
                        ).wait()

                @plsc.parallel_loop(0, D, step=1, unroll=8)
                def tbody(d):
                    didx = jnp.full((16,), 0, jnp.int32) + d
                    for c0 in range(CB // 16):
                        trs[s][d, pl.ds(c0 * 16, 16)] = plsc.load_gather(
                            rows[s], [cidx[c0], didx]
                        )

                for rb in range(RB):
                    pltpu.async_copy(
                        trs[s].at[pl.ds(rb * 8, 8)],
                        out_hbm.at[fi, rb, cb],
                        wsems[s],
                    )
            return carry

        lax.fori_loop(0, per_w // 2, outer, 0)
        for s in range(2):
            for rb in range(RB):
                pltpu.make_async_copy(
                    trs[s].at[pl.ds(rb * 8, 8)], out_hbm.at[0, rb, 0],
                    wsems[s],
                ).wait()

    return k2(xt2, wlin)


def kernel(x, weight):
    b, f = x.shape
    xt2 = x.T.astype(jnp.int32).reshape(f * (b // CB), CB)
    wtail = weight[V - 64:].reshape(16, CB)
    wlin = _detile_table(weight.T, wtail)
    y = _gather_rows((xt2, wlin.reshape(V, D)), b, f)
    return y.transpose(2, 4, 0, 1, 3).reshape(b, f, D)
